# baseline (speedup 1.0000x reference)
import functools

import jax
import jax.numpy as jnp
from jax import lax
from jax.experimental import pallas as pl
from jax.experimental.pallas import tpu as pltpu
from jax.experimental.pallas import tpu_sc as plsc

PAD_ID_K = 0
D_K = 64
SCALE_K = float(D_K) ** 0.5

NC_K = 2
NS_K = 16
NW_K = NC_K * NS_K
B_K = 4096 * 200
BPW_K = B_K // NW_K
CH_K = 256
NCHUNK_K = BPW_K // CH_K
GSUB_K = 128


def _emb_body(idx_hbm, tab_hbm, out_hbm, idx_v, rows_v, scl_v, sem):
    wid = lax.axis_index("s") * NC_K + lax.axis_index("c")
    base = wid * BPW_K
    pltpu.sync_copy(idx_hbm.at[pl.ds(base, BPW_K)], idx_v)

    def chunk_body(g, carry):
        coff = g * CH_K
        for j in range(CH_K // GSUB_K):
            pltpu.make_async_copy(
                tab_hbm.at[idx_v.at[pl.ds(coff + j * GSUB_K, GSUB_K)]],
                rows_v.at[pl.ds(j * GSUB_K, GSUB_K)],
                sem,
            ).start()
        for j in range(CH_K // GSUB_K):
            pltpu.make_async_copy(
                tab_hbm.at[idx_v.at[pl.ds(coff + j * GSUB_K, GSUB_K)]],
                rows_v.at[pl.ds(j * GSUB_K, GSUB_K)],
                sem,
            ).wait()

        def scl_body(t, c):
            iv = idx_v[pl.ds(coff + t * 16, 16)]
            scl_v[pl.ds(t * 16, 16)] = jnp.where(
                iv != PAD_ID_K, jnp.float32(SCALE_K), jnp.float32(0.0)
            )
            return c

        lax.fori_loop(0, CH_K // 16, scl_body, 0)

        def row_body(r, c):
            s = plsc.load_gather(scl_v, [jnp.zeros((16,), jnp.int32) + r])
            for j in range(D_K // 16):
                rows_v[r, pl.ds(j * 16, 16)] = rows_v[r, pl.ds(j * 16, 16)] * s
            return c

        lax.fori_loop(0, CH_K, row_body, 0)

        pltpu.sync_copy(rows_v, out_hbm.at[pl.ds(base + coff, CH_K)])
        return carry

    lax.fori_loop(0, NCHUNK_K, chunk_body, 0)


_emb = functools.partial(
    pl.kernel,
    out_type=jax.ShapeDtypeStruct((B_K, D_K), jnp.float32),
    mesh=plsc.VectorSubcoreMesh(core_axis_name="c", subcore_axis_name="s"),
    scratch_types=[
        pltpu.VMEM((BPW_K,), jnp.int32),
        pltpu.VMEM((CH_K, D_K), jnp.float32),
        pltpu.VMEM((CH_K,), jnp.float32),
        pltpu.SemaphoreType.DMA,
    ],
    compiler_params=pltpu.CompilerParams(
        needs_layout_passes=False, use_tc_tiling_on_sc=False
    ),
)(_emb_body)


@jax.jit
def kernel(input, lookup_table):
    ids = input.reshape(-1).astype(jnp.int32)
    out = _emb(ids, lookup_table)
    return out.reshape(*input.shape, D_K)


# speedup vs baseline: 1.0399x; 1.0399x over previous
import functools

import jax
import jax.numpy as jnp
from jax import lax
from jax.experimental import pallas as pl
from jax.experimental.pallas import tpu as pltpu
from jax.experimental.pallas import tpu_sc as plsc

PAD_ID_K = 0
D_K = 64
SCALE_K = float(D_K) ** 0.5

NC_K = 2
NS_K = 16
NW_K = NC_K * NS_K
B_K = 4096 * 200
BPW_K = B_K // NW_K
CH_K = 256
NCHUNK_K = BPW_K // CH_K
GSUB_K = 128
NBUF_K = 2


def _emb_body(idx_hbm, tab_hbm, out_hbm, idx_v, in_v, out_v, sem_g, sem_o):
    wid = lax.axis_index("s") * NC_K + lax.axis_index("c")
    base = wid * BPW_K
    pltpu.sync_copy(idx_hbm.at[pl.ds(base, BPW_K)], idx_v)

    def gather_cp(chunk, b, j):
        return pltpu.make_async_copy(
            tab_hbm.at[idx_v.at[pl.ds(chunk * CH_K + j * GSUB_K, GSUB_K)]],
            in_v.at[b, pl.ds(j * GSUB_K, GSUB_K)],
            sem_g.at[b],
        )

    def scatter_cp(chunk, b):
        return pltpu.make_async_copy(
            out_v.at[b],
            out_hbm.at[pl.ds(base + chunk * CH_K, CH_K)],
            sem_o.at[b],
        )

    def issue_gather(chunk, b):
        for j in range(CH_K // GSUB_K):
            gather_cp(chunk, b, j).start()

    def wait_gather(chunk, b):
        for j in range(CH_K // GSUB_K):
            gather_cp(chunk, b, j).wait()

    def compute(chunk, b):
        coff = chunk * CH_K

        def grp_body(t, c):
            row0 = t * 16
            for r in range(16):
                iv = plsc.load_gather(
                    idx_v, [jnp.zeros((16,), jnp.int32) + (coff + row0 + r)]
                )
                s = jnp.where(iv != PAD_ID_K, jnp.float32(SCALE_K), jnp.float32(0.0))
                for j in range(D_K // 16):
                    out_v[b, row0 + r, pl.ds(j * 16, 16)] = (
                        in_v[b, row0 + r, pl.ds(j * 16, 16)] * s
                    )
            return c

        lax.fori_loop(0, CH_K // 16, grp_body, 0)

    for b in range(NBUF_K):
        issue_gather(b, b)
    for b in range(NBUF_K):
        wait_gather(b, b)
        compute(b, b)
        scatter_cp(b, b).start()
        issue_gather(b + NBUF_K, b)

    def pair_body(p, carry):
        for b in range(NBUF_K):
            chunk = p * NBUF_K + b
            wait_gather(chunk, b)
            scatter_cp(chunk - NBUF_K, b).wait()
            compute(chunk, b)
            scatter_cp(chunk, b).start()
            issue_gather(chunk + NBUF_K, b)
        return carry

    lax.fori_loop(1, NCHUNK_K // NBUF_K - 1, pair_body, 0)

    for b in range(NBUF_K):
        chunk = NCHUNK_K - NBUF_K + b
        wait_gather(chunk, b)
        scatter_cp(chunk - NBUF_K, b).wait()
        compute(chunk, b)
        scatter_cp(chunk, b).start()
    for b in range(NBUF_K):
        scatter_cp(NCHUNK_K - NBUF_K + b, b).wait()


_emb = functools.partial(
    pl.kernel,
    out_type=jax.ShapeDtypeStruct((B_K, D_K), jnp.float32),
    mesh=plsc.VectorSubcoreMesh(core_axis_name="c", subcore_axis_name="s"),
    scratch_types=[
        pltpu.VMEM((BPW_K,), jnp.int32),
        pltpu.VMEM((NBUF_K, CH_K, D_K), jnp.float32),
        pltpu.VMEM((NBUF_K, CH_K, D_K), jnp.float32),
        pltpu.SemaphoreType.DMA((NBUF_K,)),
        pltpu.SemaphoreType.DMA((NBUF_K,)),
    ],
    compiler_params=pltpu.CompilerParams(
        needs_layout_passes=False, use_tc_tiling_on_sc=False
    ),
)(_emb_body)


@jax.jit
def kernel(input, lookup_table):
    ids = input.reshape(-1).astype(jnp.int32)
    out = _emb(ids, lookup_table)
    return out.reshape(*input.shape, D_K)


# speedup vs baseline: 1.1759x; 1.1308x over previous
import functools

import jax
import jax.numpy as jnp
from jax import lax
from jax.experimental import pallas as pl
from jax.experimental.pallas import tpu as pltpu
from jax.experimental.pallas import tpu_sc as plsc

PAD_ID_K = 0
D_K = 64
SCALE_K = float(D_K) ** 0.5

NC_K = 2
NS_K = 16
NW_K = NC_K * NS_K
B_K = 4096 * 200
V_K = 1000000
BPW_K = B_K // NW_K
CH_K = 128
NCHUNK_K = BPW_K // CH_K
NBUF_K = 2


def _emb_body(idx_hbm, tab_hbm, out_hbm, idx_v, pidx_v, in_v, out_v, sem_g, sem_o):
    wid = lax.axis_index("s") * NC_K + lax.axis_index("c")
    base = wid * BPW_K
    pltpu.sync_copy(idx_hbm.at[pl.ds(base, BPW_K)], idx_v)

    def pid_body(t, c):
        iv = idx_v[pl.ds(t * 16, 16)]
        pidx_v[pl.ds(t * 16, 16)] = lax.shift_right_logical(iv, 1)
        return c

    lax.fori_loop(0, BPW_K // 16, pid_body, 0)

    def gather_cp(chunk, b):
        return pltpu.make_async_copy(
            tab_hbm.at[pidx_v.at[pl.ds(chunk * CH_K, CH_K)]],
            in_v.at[b],
            sem_g.at[b],
        )

    def scatter_cp(chunk, b):
        return pltpu.make_async_copy(
            out_v.at[b],
            out_hbm.at[pl.ds(base + chunk * CH_K, CH_K)],
            sem_o.at[b],
        )

    def compute(chunk, b):
        coff = chunk * CH_K

        def grp_body(t, c):
            row0 = t * 16
            iv16 = idx_v[pl.ds(coff + row0, 16)]
            for r in range(16):
                tid = iv16[r]
                h64 = (tid & 1) * D_K
                s = jnp.where(tid != PAD_ID_K, jnp.float32(SCALE_K), jnp.float32(0.0))
                sv = jnp.zeros((16,), jnp.float32) + s
                for j in range(D_K // 16):
                    out_v[b, row0 + r, pl.ds(j * 16, 16)] = (
                        in_v[b, row0 + r, pl.ds(h64 + j * 16, 16)] * sv
                    )
            return c

        lax.fori_loop(0, CH_K // 16, grp_body, 0)

    for b in range(NBUF_K):
        gather_cp(b, b).start()
    for b in range(NBUF_K):
        gather_cp(b, b).wait()
        compute(b, b)
        scatter_cp(b, b).start()
        gather_cp(b + NBUF_K, b).start()

    def pair_body(p, carry):
        for b in range(NBUF_K):
            chunk = p * NBUF_K + b
            gather_cp(chunk, b).wait()
            scatter_cp(chunk - NBUF_K, b).wait()
            compute(chunk, b)
            scatter_cp(chunk, b).start()
            gather_cp(chunk + NBUF_K, b).start()
        return carry

    lax.fori_loop(1, NCHUNK_K // NBUF_K - 1, pair_body, 0)

    for b in range(NBUF_K):
        chunk = NCHUNK_K - NBUF_K + b
        gather_cp(chunk, b).wait()
        scatter_cp(chunk - NBUF_K, b).wait()
        compute(chunk, b)
        scatter_cp(chunk, b).start()
    for b in range(NBUF_K):
        scatter_cp(NCHUNK_K - NBUF_K + b, b).wait()


_emb = functools.partial(
    pl.kernel,
    out_type=jax.ShapeDtypeStruct((B_K, D_K), jnp.float32),
    mesh=plsc.VectorSubcoreMesh(core_axis_name="c", subcore_axis_name="s"),
    scratch_types=[
        pltpu.VMEM((BPW_K,), jnp.int32),
        pltpu.VMEM((BPW_K,), jnp.int32),
        pltpu.VMEM((NBUF_K, CH_K, 2 * D_K), jnp.float32),
        pltpu.VMEM((NBUF_K, CH_K, D_K), jnp.float32),
        pltpu.SemaphoreType.DMA((NBUF_K,)),
        pltpu.SemaphoreType.DMA((NBUF_K,)),
    ],
    compiler_params=pltpu.CompilerParams(use_tc_tiling_on_sc=True),
)(_emb_body)


@jax.jit
def kernel(input, lookup_table):
    ids = input.reshape(-1).astype(jnp.int32)
    tab2 = lookup_table.reshape(V_K // 2, 2 * D_K)
    out = _emb(ids, tab2)
    return out.reshape(*input.shape, D_K)


# speedup vs baseline: 1.3752x; 1.1695x over previous
import functools

import jax
import jax.numpy as jnp
from jax import lax
from jax.experimental import pallas as pl
from jax.experimental.pallas import tpu as pltpu
from jax.experimental.pallas import tpu_sc as plsc

PAD_ID_K = 0
D_K = 64
SCALE_K = float(D_K) ** 0.5

NC_K = 2
NS_K = 16
NW_K = NC_K * NS_K
NB_K = 4096
NJ_K = 200
V_K = 1000000
BW_K = NB_K // NW_K
NBUF_K = 2


def _emb_body(ids_hbm, tab_hbm, out_hbm, ids_v, pidx_v, scl_v, gbuf, obuf,
              sem_g, sem_o):
    wid = lax.axis_index("s") * NC_K + lax.axis_index("c")
    b0 = wid * BW_K
    pltpu.sync_copy(ids_hbm.at[:, pl.ds(b0, BW_K)], ids_v)

    iota = lax.iota(jnp.int32, 16)

    def pre_body(j, c):
        for g in range(BW_K // 16):
            iv = ids_v[j, pl.ds(g * 16, 16)]
            pidx_v[j, pl.ds(g * 16, 16)] = iv + iv
            scl_v[j, pl.ds(g * 16, 16)] = jnp.where(
                iv != PAD_ID_K, jnp.float32(SCALE_K), jnp.float32(0.0)
            )
        return c

    lax.fori_loop(0, NJ_K, pre_body, 0)

    def gather_cp(j, bb):
        return pltpu.make_async_copy(
            tab_hbm.at[pidx_v.at[j]], gbuf.at[bb], sem_g.at[bb]
        )

    def scatter_cp(j, bb):
        return pltpu.make_async_copy(
            obuf.at[bb], out_hbm.at[j, :, wid], sem_o.at[bb]
        )

    def compute(j, bb):
        bbv = jnp.zeros((16,), jnp.int32) + bb

        def blk_body(blk, c):
            bB = blk & 7
            cg = blk >> 3
            bvec = jnp.zeros((16,), jnp.int32) + (bB * 16) + iota
            sv = scl_v[j, pl.ds(bB * 16, 16)]
            for d in range(16):
                cvec = jnp.zeros((16,), jnp.int32) + (cg * 16) + ((iota + d) & 15)
                val = plsc.load_gather(gbuf, [bbv, bvec, cvec]) * sv
                plsc.store_scatter(
                    obuf,
                    [bbv, lax.shift_right_logical(cvec, 3), cvec & 7, bvec],
                    val,
                )
            return c

        lax.fori_loop(0, (BW_K // 16) * (D_K // 16), blk_body, 0)

    for bb in range(NBUF_K):
        gather_cp(bb, bb).start()
    for bb in range(NBUF_K):
        gather_cp(bb, bb).wait()
        compute(bb, bb)
        scatter_cp(bb, bb).start()
        gather_cp(bb + NBUF_K, bb).start()

    def pair_body(p, carry):
        for bb in range(NBUF_K):
            j = p * NBUF_K + bb
            gather_cp(j, bb).wait()
            scatter_cp(j - NBUF_K, bb).wait()
            compute(j, bb)
            scatter_cp(j, bb).start()
            gather_cp(j + NBUF_K, bb).start()
        return carry

    lax.fori_loop(1, NJ_K // NBUF_K - 1, pair_body, 0)

    for bb in range(NBUF_K):
        j = NJ_K - NBUF_K + bb
        gather_cp(j, bb).wait()
        scatter_cp(j - NBUF_K, bb).wait()
        compute(j, bb)
        scatter_cp(j, bb).start()
    for bb in range(NBUF_K):
        scatter_cp(NJ_K - NBUF_K + bb, bb).wait()


_emb = functools.partial(
    pl.kernel,
    out_type=jax.ShapeDtypeStruct((NJ_K, 8, NW_K, 8, 128), jnp.float32),
    mesh=plsc.VectorSubcoreMesh(core_axis_name="c", subcore_axis_name="s"),
    scratch_types=[
        pltpu.VMEM((NJ_K, BW_K), jnp.int32),
        pltpu.VMEM((NJ_K, BW_K), jnp.int32),
        pltpu.VMEM((NJ_K, BW_K), jnp.float32),
        pltpu.VMEM((NBUF_K, BW_K, D_K), jnp.float32),
        pltpu.VMEM((NBUF_K, 8, 8, 128), jnp.float32),
        pltpu.SemaphoreType.DMA((NBUF_K,)),
        pltpu.SemaphoreType.DMA((NBUF_K,)),
    ],
    compiler_params=pltpu.CompilerParams(
        needs_layout_passes=False, use_tc_tiling_on_sc=False
    ),
)(_emb_body)


@jax.jit
def kernel(input, lookup_table):
    ids_t = input.T.astype(jnp.int32)
    tabp = jnp.pad(lookup_table, ((0, 0), (0, D_K)))
    tab2 = tabp.reshape(2 * V_K, D_K)
    out5 = _emb(ids_t, tab2)
    return out5.transpose(2, 4, 0, 1, 3).reshape(NB_K, NJ_K, D_K)
